# baseline (speedup 1.0000x reference)
import jax
import jax.numpy as jnp
from jax import lax
from jax.experimental import pallas as pl
from jax.experimental.pallas import tpu as pltpu
from jax.experimental.pallas import tpu_sc as plsc

N_NODES = 10000
D = 128

NP = 10240
N_WORKERS = 32
CHUNK = 128
CHUNKS = 80
F_CHUNKS = 98
S_CHUNKS = 62
EPW = CHUNK * CHUNKS
EP = EPW * N_WORKERS
ROWS_PER_TILE = NP // 16
WB_CHUNKS = ROWS_PER_TILE // CHUNK

_SC_MESH = plsc.VectorSubcoreMesh(core_axis_name="c", subcore_axis_name="s")


def _zero_block(ref):
  zero16 = jnp.zeros((16,), jnp.float32)

  @pl.loop(0, CHUNK)
  def _z(r):
    for j in range(D // 16):
      ref[r, pl.ds(j * 16, 16)] = zero16


def _sc_agg_body(hw_hbm, src_hbm, dst_hbm, out_hbm,
                 acc, src_v, dst_v, rows_v, sem):
  c = lax.axis_index("c")
  s = lax.axis_index("s")
  nch = jnp.where(c == 0, S_CHUNKS, F_CHUNKS)
  cbase = jnp.where(c == 0, s * S_CHUNKS, 16 * S_CHUNKS + s * F_CHUNKS)

  _zero_block(rows_v)
  for k in range(WB_CHUNKS):
    pltpu.sync_copy(rows_v, acc.at[pl.ds(s * ROWS_PER_TILE + k * CHUNK, CHUNK)])

  plsc.subcore_barrier()

  @pl.loop(0, nch)
  def _edges(i):
    base = (cbase + i) * CHUNK
    pltpu.sync_copy(src_hbm.at[pl.ds(base, CHUNK)], src_v)
    pltpu.sync_copy(dst_hbm.at[pl.ds(base, CHUNK)], dst_v)
    pltpu.async_copy(hw_hbm.at[src_v], rows_v, sem).wait()
    pltpu.sync_copy(rows_v, acc.at[dst_v], add=True)

  plsc.subcore_barrier()

  for k in range(WB_CHUNKS):
    row0 = s * ROWS_PER_TILE + k * CHUNK
    pltpu.sync_copy(acc.at[pl.ds(row0, CHUNK)],
                    out_hbm.at[pl.ds(c * NP + row0, CHUNK)])


_sc_agg = pl.kernel(
    _sc_agg_body,
    out_type=jax.ShapeDtypeStruct((2 * NP, D), jnp.float32),
    mesh=_SC_MESH,
    scratch_types=(
        pltpu.VMEM_SHARED((NP, D), jnp.float32),
        pltpu.VMEM((CHUNK,), jnp.int32),
        pltpu.VMEM((CHUNK,), jnp.int32),
        pltpu.VMEM((CHUNK, D), jnp.float32),
        pltpu.SemaphoreType.DMA,
    ),
)


def _sc_deg_body(dst_hbm, deg_hbm, degacc, dst_v, ones_v):
  c = lax.axis_index("c")
  s = lax.axis_index("s")
  nch = jnp.where(c == 0, S_CHUNKS, F_CHUNKS)
  cbase = jnp.where(c == 0, s * S_CHUNKS, 16 * S_CHUNKS + s * F_CHUNKS)

  _zero_block(ones_v)
  for k in range(WB_CHUNKS):
    pltpu.sync_copy(ones_v,
                    degacc.at[pl.ds(s * ROWS_PER_TILE + k * CHUNK, CHUNK)])

  one16 = jnp.ones((16,), jnp.float32)

  @pl.loop(0, CHUNK)
  def _fill(r):
    for j in range(D // 16):
      ones_v[r, pl.ds(j * 16, 16)] = one16

  plsc.subcore_barrier()

  @pl.loop(0, nch)
  def _edges(i):
    base = (cbase + i) * CHUNK
    pltpu.sync_copy(dst_hbm.at[pl.ds(base, CHUNK)], dst_v)
    pltpu.sync_copy(ones_v, degacc.at[dst_v], add=True)

  plsc.subcore_barrier()

  for k in range(WB_CHUNKS):
    row0 = s * ROWS_PER_TILE + k * CHUNK
    pltpu.sync_copy(degacc.at[pl.ds(row0, CHUNK)],
                    deg_hbm.at[pl.ds(c * NP + row0, CHUNK)])


_sc_deg = pl.kernel(
    _sc_deg_body,
    out_type=jax.ShapeDtypeStruct((2 * NP, D), jnp.float32),
    mesh=_SC_MESH,
    scratch_types=(
        pltpu.VMEM_SHARED((NP, D), jnp.float32),
        pltpu.VMEM((CHUNK,), jnp.int32),
        pltpu.VMEM((CHUNK, D), jnp.float32),
    ),
)


_BLK = 1024


def _mm_body(h_ref, w_ref, o_ref):
  o_ref[...] = jnp.dot(h_ref[...], w_ref[...],
                       preferred_element_type=jnp.float32)


def _tc_matmul(h, w):
  return pl.pallas_call(
      _mm_body,
      grid=(NP // _BLK,),
      in_specs=[
          pl.BlockSpec((_BLK, D), lambda i: (i, 0)),
          pl.BlockSpec((D, D), lambda i: (0, 0)),
      ],
      out_specs=pl.BlockSpec((_BLK, D), lambda i: (i, 0)),
      out_shape=jax.ShapeDtypeStruct((NP, D), jnp.float32),
  )(h, w)


def _post_mm_body(p_ref, dp_ref, b_ref, w_ref, o_ref):
  agg = p_ref[0] + p_ref[1]
  deg = dp_ref[0, :, 0] + dp_ref[1, :, 0]
  deg = jnp.maximum(deg, 1.0)
  h = agg / deg[:, None] + b_ref[...]
  h = jnp.maximum(h, 0.0)
  o_ref[...] = jnp.dot(h, w_ref[...], preferred_element_type=jnp.float32)


def _tc_post_mm(p, dp, b, w):
  return pl.pallas_call(
      _post_mm_body,
      grid=(NP // _BLK,),
      in_specs=[
          pl.BlockSpec((2, _BLK, D), lambda i: (0, i, 0)),
          pl.BlockSpec((2, _BLK, D), lambda i: (0, i, 0)),
          pl.BlockSpec((D,), lambda i: (0,)),
          pl.BlockSpec((D, D), lambda i: (0, 0)),
      ],
      out_specs=pl.BlockSpec((_BLK, D), lambda i: (i, 0)),
      out_shape=jax.ShapeDtypeStruct((NP, D), jnp.float32),
  )(p, dp, b, w)


def _post_final_body(p_ref, dp_ref, b_ref, o_ref):
  agg = p_ref[0] + p_ref[1]
  deg = dp_ref[0, :, 0] + dp_ref[1, :, 0]
  deg = jnp.maximum(deg, 1.0)
  o_ref[...] = agg / deg[:, None] + b_ref[...]


def _tc_post_final(p, dp, b):
  return pl.pallas_call(
      _post_final_body,
      grid=(NP // _BLK,),
      in_specs=[
          pl.BlockSpec((2, _BLK, D), lambda i: (0, i, 0)),
          pl.BlockSpec((2, _BLK, D), lambda i: (0, i, 0)),
          pl.BlockSpec((D,), lambda i: (0,)),
      ],
      out_specs=pl.BlockSpec((_BLK, D), lambda i: (i, 0)),
      out_shape=jax.ShapeDtypeStruct((NP, D), jnp.float32),
  )(p, dp, b)


@jax.jit
def kernel(features, edge_index, W1, b1, W2, b2, W3, b3):
  src = edge_index[0].astype(jnp.int32)
  dst = edge_index[1].astype(jnp.int32)
  n_extra = EP - src.shape[0]
  pad_idx = jnp.full((n_extra,), N_NODES, jnp.int32)
  src = jnp.concatenate([src, pad_idx])
  dst = jnp.concatenate([dst, pad_idx])

  feat_pad = jnp.zeros((NP, D), features.dtype).at[:N_NODES].set(features)

  dp = _sc_deg(dst).reshape(2, NP, D)
  hw = _tc_matmul(feat_pad, W1)
  p = _sc_agg(hw, src, dst).reshape(2, NP, D)
  hw = _tc_post_mm(p, dp, b1, W2)
  p = _sc_agg(hw, src, dst).reshape(2, NP, D)
  hw = _tc_post_mm(p, dp, b2, W3)
  p = _sc_agg(hw, src, dst).reshape(2, NP, D)
  out = _tc_post_final(p, dp, b3)
  return out[:N_NODES]


# speedup vs baseline: 1.0854x; 1.0854x over previous
import jax
import jax.numpy as jnp
from jax import lax
from jax.experimental import pallas as pl
from jax.experimental.pallas import tpu as pltpu
from jax.experimental.pallas import tpu_sc as plsc

N_NODES = 10000
D = 128

NP = 10240
N_WORKERS = 32
CHUNK = 128
CHUNKS = 80
F_CHUNKS = 80
S_CHUNKS = 80
EPW = CHUNK * CHUNKS
EP = EPW * N_WORKERS
ROWS_PER_TILE = NP // 16
WB_CHUNKS = ROWS_PER_TILE // CHUNK

_SC_MESH = plsc.VectorSubcoreMesh(core_axis_name="c", subcore_axis_name="s")


def _zero_block(ref):
  zero16 = jnp.zeros((16,), jnp.float32)

  @pl.loop(0, CHUNK)
  def _z(r):
    for j in range(D // 16):
      ref[r, pl.ds(j * 16, 16)] = zero16


def _sc_agg_body(hw_hbm, src_hbm, dst_hbm, out_hbm,
                 acc, src_v, dst_v, rows_v, sem):
  c = lax.axis_index("c")
  s = lax.axis_index("s")
  nch = jnp.where(c == 0, S_CHUNKS, F_CHUNKS)
  cbase = jnp.where(c == 0, s * S_CHUNKS, 16 * S_CHUNKS + s * F_CHUNKS)

  _zero_block(rows_v)
  for k in range(WB_CHUNKS):
    pltpu.sync_copy(rows_v, acc.at[pl.ds(s * ROWS_PER_TILE + k * CHUNK, CHUNK)])

  plsc.subcore_barrier()

  @pl.loop(0, nch)
  def _edges(i):
    base = (cbase + i) * CHUNK
    pltpu.sync_copy(src_hbm.at[pl.ds(base, CHUNK)], src_v)
    pltpu.sync_copy(dst_hbm.at[pl.ds(base, CHUNK)], dst_v)
    pltpu.async_copy(hw_hbm.at[src_v], rows_v, sem).wait()
    pltpu.sync_copy(rows_v, acc.at[dst_v], add=True)

  plsc.subcore_barrier()

  for k in range(WB_CHUNKS):
    row0 = s * ROWS_PER_TILE + k * CHUNK
    pltpu.sync_copy(acc.at[pl.ds(row0, CHUNK)],
                    out_hbm.at[pl.ds(c * NP + row0, CHUNK)])


_sc_agg = pl.kernel(
    _sc_agg_body,
    out_type=jax.ShapeDtypeStruct((2 * NP, D), jnp.float32),
    mesh=_SC_MESH,
    scratch_types=(
        pltpu.VMEM_SHARED((NP, D), jnp.float32),
        pltpu.VMEM((CHUNK,), jnp.int32),
        pltpu.VMEM((CHUNK,), jnp.int32),
        pltpu.VMEM((CHUNK, D), jnp.float32),
        pltpu.SemaphoreType.DMA,
    ),
)


def _sc_deg_body(dst_hbm, deg_hbm, degacc, dst_v, ones_v):
  c = lax.axis_index("c")
  s = lax.axis_index("s")
  nch = jnp.where(c == 0, S_CHUNKS, F_CHUNKS)
  cbase = jnp.where(c == 0, s * S_CHUNKS, 16 * S_CHUNKS + s * F_CHUNKS)

  _zero_block(ones_v)
  for k in range(WB_CHUNKS):
    pltpu.sync_copy(ones_v,
                    degacc.at[pl.ds(s * ROWS_PER_TILE + k * CHUNK, CHUNK)])

  one16 = jnp.ones((16,), jnp.float32)

  @pl.loop(0, CHUNK)
  def _fill(r):
    for j in range(D // 16):
      ones_v[r, pl.ds(j * 16, 16)] = one16

  plsc.subcore_barrier()

  @pl.loop(0, nch)
  def _edges(i):
    base = (cbase + i) * CHUNK
    pltpu.sync_copy(dst_hbm.at[pl.ds(base, CHUNK)], dst_v)
    pltpu.sync_copy(ones_v, degacc.at[dst_v], add=True)

  plsc.subcore_barrier()

  for k in range(WB_CHUNKS):
    row0 = s * ROWS_PER_TILE + k * CHUNK
    pltpu.sync_copy(degacc.at[pl.ds(row0, CHUNK)],
                    deg_hbm.at[pl.ds(c * NP + row0, CHUNK)])


_sc_deg = pl.kernel(
    _sc_deg_body,
    out_type=jax.ShapeDtypeStruct((2 * NP, D), jnp.float32),
    mesh=_SC_MESH,
    scratch_types=(
        pltpu.VMEM_SHARED((NP, D), jnp.float32),
        pltpu.VMEM((CHUNK,), jnp.int32),
        pltpu.VMEM((CHUNK, D), jnp.float32),
    ),
)


_BLK = 1024


def _mm_body(h_ref, w_ref, o_ref):
  o_ref[...] = jnp.dot(h_ref[...], w_ref[...],
                       preferred_element_type=jnp.float32)


def _tc_matmul(h, w):
  return pl.pallas_call(
      _mm_body,
      grid=(NP // _BLK,),
      in_specs=[
          pl.BlockSpec((_BLK, D), lambda i: (i, 0)),
          pl.BlockSpec((D, D), lambda i: (0, 0)),
      ],
      out_specs=pl.BlockSpec((_BLK, D), lambda i: (i, 0)),
      out_shape=jax.ShapeDtypeStruct((NP, D), jnp.float32),
  )(h, w)


def _post_mm_body(p_ref, dp_ref, b_ref, w_ref, o_ref):
  agg = p_ref[0] + p_ref[1]
  deg = dp_ref[0, :, 0] + dp_ref[1, :, 0]
  deg = jnp.maximum(deg, 1.0)
  h = agg / deg[:, None] + b_ref[...]
  h = jnp.maximum(h, 0.0)
  o_ref[...] = jnp.dot(h, w_ref[...], preferred_element_type=jnp.float32)


def _tc_post_mm(p, dp, b, w):
  return pl.pallas_call(
      _post_mm_body,
      grid=(NP // _BLK,),
      in_specs=[
          pl.BlockSpec((2, _BLK, D), lambda i: (0, i, 0)),
          pl.BlockSpec((2, _BLK, D), lambda i: (0, i, 0)),
          pl.BlockSpec((D,), lambda i: (0,)),
          pl.BlockSpec((D, D), lambda i: (0, 0)),
      ],
      out_specs=pl.BlockSpec((_BLK, D), lambda i: (i, 0)),
      out_shape=jax.ShapeDtypeStruct((NP, D), jnp.float32),
  )(p, dp, b, w)


def _post_final_body(p_ref, dp_ref, b_ref, o_ref):
  agg = p_ref[0] + p_ref[1]
  deg = dp_ref[0, :, 0] + dp_ref[1, :, 0]
  deg = jnp.maximum(deg, 1.0)
  o_ref[...] = agg / deg[:, None] + b_ref[...]


def _tc_post_final(p, dp, b):
  return pl.pallas_call(
      _post_final_body,
      grid=(NP // _BLK,),
      in_specs=[
          pl.BlockSpec((2, _BLK, D), lambda i: (0, i, 0)),
          pl.BlockSpec((2, _BLK, D), lambda i: (0, i, 0)),
          pl.BlockSpec((D,), lambda i: (0,)),
      ],
      out_specs=pl.BlockSpec((_BLK, D), lambda i: (i, 0)),
      out_shape=jax.ShapeDtypeStruct((NP, D), jnp.float32),
  )(p, dp, b)


@jax.jit
def kernel(features, edge_index, W1, b1, W2, b2, W3, b3):
  src = edge_index[0].astype(jnp.int32)
  dst = edge_index[1].astype(jnp.int32)
  n_extra = EP - src.shape[0]
  pad_idx = jnp.full((n_extra,), N_NODES, jnp.int32)
  src = jnp.concatenate([src, pad_idx])
  dst = jnp.concatenate([dst, pad_idx])

  feat_pad = jnp.zeros((NP, D), features.dtype).at[:N_NODES].set(features)

  dp = _sc_deg(dst).reshape(2, NP, D)
  hw = _tc_matmul(feat_pad, W1)
  p = _sc_agg(hw, src, dst).reshape(2, NP, D)
  hw = _tc_post_mm(p, dp, b1, W2)
  p = _sc_agg(hw, src, dst).reshape(2, NP, D)
  hw = _tc_post_mm(p, dp, b2, W3)
  p = _sc_agg(hw, src, dst).reshape(2, NP, D)
  out = _tc_post_final(p, dp, b3)
  return out[:N_NODES]


# speedup vs baseline: 1.0863x; 1.0009x over previous
import jax
import jax.numpy as jnp
from jax import lax
from jax.experimental import pallas as pl
from jax.experimental.pallas import tpu as pltpu
from jax.experimental.pallas import tpu_sc as plsc

N_NODES = 10000
D = 128

NP = 10240
N_WORKERS = 32
CHUNK = 128
CHUNKS = 80
F_CHUNKS = 80
S_CHUNKS = 80
EPW = CHUNK * CHUNKS
EP = EPW * N_WORKERS
ROWS_PER_TILE = NP // 16
WB_CHUNKS = ROWS_PER_TILE // CHUNK

_SC_MESH = plsc.VectorSubcoreMesh(core_axis_name="c", subcore_axis_name="s")


def _zero_block(ref):
  zero16 = jnp.zeros((16,), jnp.float32)

  @pl.loop(0, CHUNK)
  def _z(r):
    for j in range(D // 16):
      ref[r, pl.ds(j * 16, 16)] = zero16


def _sc_agg_body(hw_hbm, src_hbm, dst_hbm, out_hbm,
                 acc, src_v, dst_v, rows_v, sem):
  c = lax.axis_index("c")
  s = lax.axis_index("s")
  wid = s * 2 + c
  cbase = wid * CHUNKS

  _zero_block(rows_v)
  for k in range(WB_CHUNKS):
    pltpu.sync_copy(rows_v, acc.at[pl.ds(s * ROWS_PER_TILE + k * CHUNK, CHUNK)])

  plsc.subcore_barrier()

  @pl.loop(0, CHUNKS)
  def _edges(i):
    base = (cbase + i) * CHUNK
    pltpu.sync_copy(src_hbm.at[pl.ds(base, CHUNK)], src_v)
    pltpu.sync_copy(dst_hbm.at[pl.ds(base, CHUNK)], dst_v)
    pltpu.async_copy(hw_hbm.at[src_v], rows_v, sem).wait()
    pltpu.sync_copy(rows_v, acc.at[dst_v], add=True)

  plsc.subcore_barrier()

  for k in range(WB_CHUNKS):
    row0 = s * ROWS_PER_TILE + k * CHUNK
    pltpu.sync_copy(acc.at[pl.ds(row0, CHUNK)],
                    out_hbm.at[pl.ds(c * NP + row0, CHUNK)])


_sc_agg = pl.kernel(
    _sc_agg_body,
    out_type=jax.ShapeDtypeStruct((2 * NP, D), jnp.float32),
    mesh=_SC_MESH,
    scratch_types=(
        pltpu.VMEM_SHARED((NP, D), jnp.float32),
        pltpu.VMEM((CHUNK,), jnp.int32),
        pltpu.VMEM((CHUNK,), jnp.int32),
        pltpu.VMEM((CHUNK, D), jnp.float32),
        pltpu.SemaphoreType.DMA,
    ),
)


def _sc_deg_body(dst_hbm, deg_hbm, degacc, dst_v, ones_v):
  c = lax.axis_index("c")
  s = lax.axis_index("s")
  wid = s * 2 + c
  cbase = wid * CHUNKS

  _zero_block(ones_v)
  for k in range(WB_CHUNKS):
    pltpu.sync_copy(ones_v,
                    degacc.at[pl.ds(s * ROWS_PER_TILE + k * CHUNK, CHUNK)])

  one16 = jnp.ones((16,), jnp.float32)

  @pl.loop(0, CHUNK)
  def _fill(r):
    for j in range(D // 16):
      ones_v[r, pl.ds(j * 16, 16)] = one16

  plsc.subcore_barrier()

  @pl.loop(0, CHUNKS)
  def _edges(i):
    base = (cbase + i) * CHUNK
    pltpu.sync_copy(dst_hbm.at[pl.ds(base, CHUNK)], dst_v)
    pltpu.sync_copy(ones_v, degacc.at[dst_v], add=True)

  plsc.subcore_barrier()

  for k in range(WB_CHUNKS):
    row0 = s * ROWS_PER_TILE + k * CHUNK
    pltpu.sync_copy(degacc.at[pl.ds(row0, CHUNK)],
                    deg_hbm.at[pl.ds(c * NP + row0, CHUNK)])


_sc_deg = pl.kernel(
    _sc_deg_body,
    out_type=jax.ShapeDtypeStruct((2 * NP, D), jnp.float32),
    mesh=_SC_MESH,
    scratch_types=(
        pltpu.VMEM_SHARED((NP, D), jnp.float32),
        pltpu.VMEM((CHUNK,), jnp.int32),
        pltpu.VMEM((CHUNK, D), jnp.float32),
    ),
)


_BLK = 1024


def _mm_body(h_ref, w_ref, o_ref):
  o_ref[...] = jnp.dot(h_ref[...], w_ref[...],
                       preferred_element_type=jnp.float32)


def _tc_matmul(h, w):
  return pl.pallas_call(
      _mm_body,
      grid=(NP // _BLK,),
      in_specs=[
          pl.BlockSpec((_BLK, D), lambda i: (i, 0)),
          pl.BlockSpec((D, D), lambda i: (0, 0)),
      ],
      out_specs=pl.BlockSpec((_BLK, D), lambda i: (i, 0)),
      out_shape=jax.ShapeDtypeStruct((NP, D), jnp.float32),
  )(h, w)


def _post_mm_body(p_ref, dp_ref, b_ref, w_ref, o_ref):
  agg = p_ref[0] + p_ref[1]
  deg = dp_ref[0, :, 0] + dp_ref[1, :, 0]
  deg = jnp.maximum(deg, 1.0)
  h = agg / deg[:, None] + b_ref[...]
  h = jnp.maximum(h, 0.0)
  o_ref[...] = jnp.dot(h, w_ref[...], preferred_element_type=jnp.float32)


def _tc_post_mm(p, dp, b, w):
  return pl.pallas_call(
      _post_mm_body,
      grid=(NP // _BLK,),
      in_specs=[
          pl.BlockSpec((2, _BLK, D), lambda i: (0, i, 0)),
          pl.BlockSpec((2, _BLK, D), lambda i: (0, i, 0)),
          pl.BlockSpec((D,), lambda i: (0,)),
          pl.BlockSpec((D, D), lambda i: (0, 0)),
      ],
      out_specs=pl.BlockSpec((_BLK, D), lambda i: (i, 0)),
      out_shape=jax.ShapeDtypeStruct((NP, D), jnp.float32),
  )(p, dp, b, w)


def _post_final_body(p_ref, dp_ref, b_ref, o_ref):
  agg = p_ref[0] + p_ref[1]
  deg = dp_ref[0, :, 0] + dp_ref[1, :, 0]
  deg = jnp.maximum(deg, 1.0)
  o_ref[...] = agg / deg[:, None] + b_ref[...]


def _tc_post_final(p, dp, b):
  return pl.pallas_call(
      _post_final_body,
      grid=(NP // _BLK,),
      in_specs=[
          pl.BlockSpec((2, _BLK, D), lambda i: (0, i, 0)),
          pl.BlockSpec((2, _BLK, D), lambda i: (0, i, 0)),
          pl.BlockSpec((D,), lambda i: (0,)),
      ],
      out_specs=pl.BlockSpec((_BLK, D), lambda i: (i, 0)),
      out_shape=jax.ShapeDtypeStruct((NP, D), jnp.float32),
  )(p, dp, b)


@jax.jit
def kernel(features, edge_index, W1, b1, W2, b2, W3, b3):
  src = edge_index[0].astype(jnp.int32)
  dst = edge_index[1].astype(jnp.int32)
  n_extra = EP - src.shape[0]
  pad_idx = jnp.full((n_extra,), N_NODES, jnp.int32)
  src = jnp.concatenate([src, pad_idx])
  dst = jnp.concatenate([dst, pad_idx])

  feat_pad = jnp.zeros((NP, D), features.dtype).at[:N_NODES].set(features)

  dp = _sc_deg(dst).reshape(2, NP, D)
  hw = _tc_matmul(feat_pad, W1)
  p = _sc_agg(hw, src, dst).reshape(2, NP, D)
  hw = _tc_post_mm(p, dp, b1, W2)
  p = _sc_agg(hw, src, dst).reshape(2, NP, D)
  hw = _tc_post_mm(p, dp, b2, W3)
  p = _sc_agg(hw, src, dst).reshape(2, NP, D)
  out = _tc_post_final(p, dp, b3)
  return out[:N_NODES]


# speedup vs baseline: 1.4801x; 1.3625x over previous
import jax
import jax.numpy as jnp
from jax import lax
from jax.experimental import pallas as pl
from jax.experimental.pallas import tpu as pltpu
from jax.experimental.pallas import tpu_sc as plsc

N_NODES = 10000
D = 128

NP = 10240
N_WORKERS = 32
CHUNK = 128
CHUNKS = 79
F_CHUNKS = 79
S_CHUNKS = 79
EPW = CHUNK * CHUNKS
EP = EPW * N_WORKERS
ROWS_PER_TILE = NP // 16
WB_CHUNKS = ROWS_PER_TILE // CHUNK

_SC_MESH = plsc.VectorSubcoreMesh(core_axis_name="c", subcore_axis_name="s")


def _zero_block(ref):
  zero16 = jnp.zeros((16,), jnp.float32)

  @pl.loop(0, CHUNK)
  def _z(r):
    for j in range(D // 16):
      ref[r, pl.ds(j * 16, 16)] = zero16


def _sc_agg_body(hw_hbm, src_hbm, dst_hbm, out_hbm,
                 acc, src_v, dst_v, rows_v, sem):
  c = lax.axis_index("c")
  s = lax.axis_index("s")
  wid = s * 2 + c
  cbase = wid * CHUNKS

  _zero_block(rows_v)
  for k in range(WB_CHUNKS):
    pltpu.sync_copy(rows_v, acc.at[pl.ds(s * ROWS_PER_TILE + k * CHUNK, CHUNK)])

  plsc.subcore_barrier()

  @pl.loop(0, CHUNKS)
  def _edges(i):
    base = (cbase + i) * CHUNK
    pltpu.sync_copy(src_hbm.at[pl.ds(base, CHUNK)], src_v)
    pltpu.sync_copy(dst_hbm.at[pl.ds(base, CHUNK)], dst_v)
    pltpu.async_copy(hw_hbm.at[src_v], rows_v, sem).wait()
    pltpu.sync_copy(rows_v, acc.at[dst_v], add=True)

  plsc.subcore_barrier()

  for k in range(WB_CHUNKS):
    row0 = s * ROWS_PER_TILE + k * CHUNK
    pltpu.sync_copy(acc.at[pl.ds(row0, CHUNK)],
                    out_hbm.at[pl.ds(c * NP + row0, CHUNK)])


_sc_agg = pl.kernel(
    _sc_agg_body,
    out_type=jax.ShapeDtypeStruct((2 * NP, D), jnp.float32),
    mesh=_SC_MESH,
    scratch_types=(
        pltpu.VMEM_SHARED((NP, D), jnp.float32),
        pltpu.VMEM((CHUNK,), jnp.int32),
        pltpu.VMEM((CHUNK,), jnp.int32),
        pltpu.VMEM((CHUNK, D), jnp.float32),
        pltpu.SemaphoreType.DMA,
    ),
)


def _sc_deg_body(dst_hbm, deg_hbm, degacc, dst_v, ones_v):
  c = lax.axis_index("c")
  s = lax.axis_index("s")
  wid = s * 2 + c
  cbase = wid * CHUNKS

  _zero_block(ones_v)
  for k in range(WB_CHUNKS):
    pltpu.sync_copy(ones_v,
                    degacc.at[pl.ds(s * ROWS_PER_TILE + k * CHUNK, CHUNK)])

  one16 = jnp.ones((16,), jnp.float32)

  @pl.loop(0, CHUNK)
  def _fill(r):
    for j in range(D // 16):
      ones_v[r, pl.ds(j * 16, 16)] = one16

  plsc.subcore_barrier()

  @pl.loop(0, CHUNKS)
  def _edges(i):
    base = (cbase + i) * CHUNK
    pltpu.sync_copy(dst_hbm.at[pl.ds(base, CHUNK)], dst_v)
    pltpu.sync_copy(ones_v, degacc.at[dst_v], add=True)

  plsc.subcore_barrier()

  for k in range(WB_CHUNKS):
    row0 = s * ROWS_PER_TILE + k * CHUNK
    pltpu.sync_copy(degacc.at[pl.ds(row0, CHUNK)],
                    deg_hbm.at[pl.ds(c * NP + row0, CHUNK)])


_sc_deg = pl.kernel(
    _sc_deg_body,
    out_type=jax.ShapeDtypeStruct((2 * NP, D), jnp.float32),
    mesh=_SC_MESH,
    scratch_types=(
        pltpu.VMEM_SHARED((NP, D), jnp.float32),
        pltpu.VMEM((CHUNK,), jnp.int32),
        pltpu.VMEM((CHUNK, D), jnp.float32),
    ),
)


_BLK = 1024


def _mm_body(h_ref, w_ref, o_ref):
  o_ref[...] = jnp.dot(h_ref[...], w_ref[...],
                       preferred_element_type=jnp.float32)


def _tc_matmul(h, w):
  return pl.pallas_call(
      _mm_body,
      grid=(NP // _BLK,),
      in_specs=[
          pl.BlockSpec((_BLK, D), lambda i: (i, 0)),
          pl.BlockSpec((D, D), lambda i: (0, 0)),
      ],
      out_specs=pl.BlockSpec((_BLK, D), lambda i: (i, 0)),
      out_shape=jax.ShapeDtypeStruct((NP, D), jnp.float32),
  )(h, w)


def _post_mm_body(p_ref, dp_ref, b_ref, w_ref, o_ref):
  agg = p_ref[0] + p_ref[1]
  deg = dp_ref[0, :, 0] + dp_ref[1, :, 0]
  deg = jnp.maximum(deg, 1.0)
  h = agg / deg[:, None] + b_ref[...]
  h = jnp.maximum(h, 0.0)
  o_ref[...] = jnp.dot(h, w_ref[...], preferred_element_type=jnp.float32)


def _tc_post_mm(p, dp, b, w):
  return pl.pallas_call(
      _post_mm_body,
      grid=(NP // _BLK,),
      in_specs=[
          pl.BlockSpec((2, _BLK, D), lambda i: (0, i, 0)),
          pl.BlockSpec((2, _BLK, D), lambda i: (0, i, 0)),
          pl.BlockSpec((D,), lambda i: (0,)),
          pl.BlockSpec((D, D), lambda i: (0, 0)),
      ],
      out_specs=pl.BlockSpec((_BLK, D), lambda i: (i, 0)),
      out_shape=jax.ShapeDtypeStruct((NP, D), jnp.float32),
  )(p, dp, b, w)


def _post_final_body(p_ref, dp_ref, b_ref, o_ref):
  agg = p_ref[0] + p_ref[1]
  deg = dp_ref[0, :, 0] + dp_ref[1, :, 0]
  deg = jnp.maximum(deg, 1.0)
  o_ref[...] = agg / deg[:, None] + b_ref[...]


def _tc_post_final(p, dp, b):
  return pl.pallas_call(
      _post_final_body,
      grid=(NP // _BLK,),
      in_specs=[
          pl.BlockSpec((2, _BLK, D), lambda i: (0, i, 0)),
          pl.BlockSpec((2, _BLK, D), lambda i: (0, i, 0)),
          pl.BlockSpec((D,), lambda i: (0,)),
      ],
      out_specs=pl.BlockSpec((_BLK, D), lambda i: (i, 0)),
      out_shape=jax.ShapeDtypeStruct((NP, D), jnp.float32),
  )(p, dp, b)


@jax.jit
def kernel(features, edge_index, W1, b1, W2, b2, W3, b3):
  src = edge_index[0].astype(jnp.int32)
  dst = edge_index[1].astype(jnp.int32)
  n_extra = EP - src.shape[0]
  pad_idx = jnp.full((n_extra,), N_NODES, jnp.int32)
  src = jnp.concatenate([src, pad_idx])
  dst = jnp.concatenate([dst, pad_idx])

  feat_pad = jnp.zeros((NP, D), features.dtype).at[:N_NODES].set(features)

  dp = _sc_deg(dst).reshape(2, NP, D)
  hw = _tc_matmul(feat_pad, W1)
  p = _sc_agg(hw, src, dst).reshape(2, NP, D)
  hw = _tc_post_mm(p, dp, b1, W2)
  p = _sc_agg(hw, src, dst).reshape(2, NP, D)
  hw = _tc_post_mm(p, dp, b2, W3)
  p = _sc_agg(hw, src, dst).reshape(2, NP, D)
  out = _tc_post_final(p, dp, b3)
  return out[:N_NODES]
